# Initial kernel scaffold; baseline (speedup 1.0000x reference)
import jax
import jax.numpy as jnp
from jax.experimental import pallas as pl


def kernel(w1_taps, b1, w2_taps, b2, S1, G, fc1_b, fc2_wT, fc2_b, fc3_wT, fc3_b, x):
    raise NotImplementedError("write your pallas kernel here")


# speedup vs baseline: 2.6189x; 2.6189x over previous
import jax
import jax.numpy as jnp
from jax import lax
from jax.experimental import pallas as pl
from jax.experimental.pallas import tpu as pltpu

C_IN, C1, C2 = 3, 6, 16
IMG, K = 32, 5
OH1 = IMG - K + 1
W1 = (OH1 - 1) * IMG + OH1
P1 = (W1 - 1) - IMG
POOL1 = 14
OH2 = POOL1 - K + 1
W2 = (OH2 - 1) * POOL1 + OH2
P2 = (W2 - 1) - POOL1
N1, N2, N3 = 120, 84, 10

NB = 64


def _lenet_block_kernel(w1_ref, b1_ref, w2_ref, b2_ref, x_ref, s1_ref, g_ref,
                        fc1b_ref, fc2w_ref, fc2b_ref, fc3w_ref, fc3b_ref,
                        o_ref, p1_ref):
    xv = [x_ref[:, ci, :] for ci in range(C_IN)]

    def c1_tap(t, accs):
        kh = t // K
        s = kh * IMG + (t - kh * K)
        sh = (IMG * IMG - s) % (IMG * IMG)
        xs = [pltpu.roll(xv[ci], sh, 1) for ci in range(C_IN)]
        return tuple(
            accs[co] + sum(w1_ref[t, co, ci] * xs[ci] for ci in range(C_IN))
            for co in range(C1))

    init1 = tuple(jnp.zeros((NB, IMG * IMG), jnp.float32) for _ in range(C1))
    y1 = lax.fori_loop(0, K * K, c1_tap, init1)

    for co in range(C1):
        y = jnp.maximum(y1[co][:, :W1] + b1_ref[co, 0], 0.0)
        m = jnp.maximum(y[:, :-1], y[:, 1:])
        m = jnp.maximum(m[:, :P1], m[:, IMG:IMG + P1])
        p1_ref[co, :, :POOL1 * POOL1] = jnp.dot(
            m, s1_ref[...], preferred_element_type=jnp.float32)

    pv = [p1_ref[ci] for ci in range(C1)]

    def c2_tap(t, accs):
        kh = t // K
        s = kh * POOL1 + (t - kh * K)
        sh = (256 - s) % 256
        ps = [pltpu.roll(pv[ci], sh, 1) for ci in range(C1)]
        return tuple(
            accs[co] + sum(w2_ref[t, co, ci] * ps[ci] for ci in range(C1))
            for co in range(C2))

    init2 = tuple(jnp.zeros((NB, 256), jnp.float32) for _ in range(C2))
    y2 = lax.fori_loop(0, K * K, c2_tap, init2)

    h1 = None
    for co in range(C2):
        y = jnp.maximum(y2[co][:, :W2] + b2_ref[co, 0], 0.0)
        m2 = jnp.maximum(y[:, :-1], y[:, 1:])
        m2 = jnp.maximum(m2[:, :P2], m2[:, POOL1:POOL1 + P2])
        d = jnp.dot(m2, g_ref[co], preferred_element_type=jnp.float32)
        h1 = d if h1 is None else h1 + d
    h1 = jnp.maximum(h1 + fc1b_ref[...], 0.0)

    h2 = jnp.maximum(
        jnp.dot(h1, fc2w_ref[...], preferred_element_type=jnp.float32)
        + fc2b_ref[...], 0.0)
    o_ref[...] = jnp.dot(h2, fc3w_ref[...],
                         preferred_element_type=jnp.float32) + fc3b_ref[...]


@jax.jit
def _lenet_forward(w1_taps, b1, w2_taps, b2, S1, G, fc1_b, fc2_wT, fc2_b,
                   fc3_wT, fc3_b, x):
    B = x.shape[0]
    x_flat = x.reshape(B, C_IN, IMG * IMG)

    def vconst(a):
        return pl.BlockSpec(a.shape, lambda b, _n=a.ndim: (0,) * _n)

    smem = pl.BlockSpec(memory_space=pltpu.SMEM)

    out = pl.pallas_call(
        _lenet_block_kernel,
        out_shape=jax.ShapeDtypeStruct((B, N3), jnp.float32),
        grid=(B // NB,),
        in_specs=[
            smem,
            smem,
            smem,
            smem,
            pl.BlockSpec((NB, C_IN, IMG * IMG), lambda b: (b, 0, 0)),
            vconst(S1), vconst(G), vconst(fc1_b), vconst(fc2_wT),
            vconst(fc2_b), vconst(fc3_wT), vconst(fc3_b),
        ],
        out_specs=pl.BlockSpec((NB, N3), lambda b: (b, 0)),
        scratch_shapes=[pltpu.VMEM((C1, NB, 256), jnp.float32)],
        compiler_params=pltpu.CompilerParams(
            dimension_semantics=("parallel",)),
    )(w1_taps, b1, w2_taps, b2, x_flat, S1, G, fc1_b, fc2_wT, fc2_b,
      fc3_wT, fc3_b)
    return out


def kernel(w1_taps, b1, w2_taps, b2, S1, G, fc1_b, fc2_wT, fc2_b, fc3_wT,
           fc3_b, x):
    return _lenet_forward(w1_taps, b1, w2_taps, b2, S1, G, fc1_b, fc2_wT,
                          fc2_b, fc3_wT, fc3_b, x)


# speedup vs baseline: 20.3281x; 7.7620x over previous
import numpy as np
import jax
import jax.numpy as jnp
from jax.experimental import pallas as pl
from jax.experimental.pallas import tpu as pltpu

C_IN, C1, C2 = 3, 6, 16
IMG, K = 32, 5
OH1 = IMG - K + 1
W1 = (OH1 - 1) * IMG + OH1
P1 = (W1 - 1) - IMG
POOL1 = 14
OH2 = POOL1 - K + 1
W2 = (OH2 - 1) * POOL1 + OH2
P2 = (W2 - 1) - POOL1
N1, N2, N3 = 120, 84, 10

NB = 256
CH = 120
NCH = 8
KIN = CH + (K - 1) * IMG + K - 1 + 1

_E1 = np.zeros((K * K, KIN, CH), np.float32)
_E2 = np.zeros((K * K, POOL1 * POOL1, W2), np.float32)
for _t in range(K * K):
    _kh, _kw = divmod(_t, K)
    _o1 = _kh * IMG + _kw
    _o2 = _kh * POOL1 + _kw
    for _q in range(CH):
        if _q + _o1 < KIN:
            _E1[_t, _q + _o1, _q] = 1.0
    for _q in range(W2):
        _E2[_t, _q + _o2, _q] = 1.0


def _lenet_block_kernel(b1_ref, b2_ref, x_ref, a1_ref, a2_ref, s1_ref, g_ref,
                        fc1b_ref, fc2w_ref, fc2b_ref, fc3w_ref, fc3b_ref,
                        o_ref):

    ys = []
    for j in range(NCH):
        lo = CH * j
        hi = min(lo + KIN, IMG * IMG)
        kw_ = hi - lo
        acc = None
        for ci in range(C_IN):
            d = jnp.dot(x_ref[:, ci, lo:hi], a1_ref[ci, :kw_, :],
                        preferred_element_type=jnp.float32)
            acc = d if acc is None else acc + d
        ys.append(acc)

    p1 = []
    for co in range(C1):
        y = jnp.concatenate([ys[j][:, co * CH:(co + 1) * CH]
                             for j in range(NCH)], axis=1)
        y = jnp.maximum(y[:, :W1] + b1_ref[co, 0], 0.0)
        m = jnp.maximum(y[:, :-1], y[:, 1:])
        m = jnp.maximum(m[:, :P1], m[:, IMG:IMG + P1])
        p1.append(jnp.dot(m, s1_ref[...],
                          preferred_element_type=jnp.float32))

    y2 = None
    for ci in range(C1):
        d = jnp.dot(p1[ci], a2_ref[ci], preferred_element_type=jnp.float32)
        y2 = d if y2 is None else y2 + d

    h1 = None
    for co in range(C2):
        y = jnp.maximum(y2[:, co * W2:(co + 1) * W2] + b2_ref[co, 0], 0.0)
        m2 = jnp.maximum(y[:, :-1], y[:, 1:])
        m2 = jnp.maximum(m2[:, :P2], m2[:, POOL1:POOL1 + P2])
        d = jnp.dot(m2, g_ref[co], preferred_element_type=jnp.float32)
        h1 = d if h1 is None else h1 + d
    h1 = jnp.maximum(h1 + fc1b_ref[...], 0.0)

    h2 = jnp.maximum(
        jnp.dot(h1, fc2w_ref[...], preferred_element_type=jnp.float32)
        + fc2b_ref[...], 0.0)
    o_ref[...] = jnp.dot(h2, fc3w_ref[...],
                         preferred_element_type=jnp.float32) + fc3b_ref[...]


@jax.jit
def _lenet_forward(w1_taps, b1, w2_taps, b2, S1, G, fc1_b, fc2_wT, fc2_b,
                   fc3_wT, fc3_b, x):
    B = x.shape[0]
    x_flat = x.reshape(B, C_IN, IMG * IMG)

    a1 = jnp.einsum("tkq,toc->ckoq", jnp.asarray(_E1),
                    w1_taps).reshape(C_IN, KIN, C1 * CH)
    a2 = jnp.einsum("tkq,toc->ckoq", jnp.asarray(_E2),
                    w2_taps).reshape(C1, POOL1 * POOL1, C2 * W2)

    def vconst(a):
        return pl.BlockSpec(a.shape, lambda b, _n=a.ndim: (0,) * _n)

    smem = pl.BlockSpec(memory_space=pltpu.SMEM)

    out = pl.pallas_call(
        _lenet_block_kernel,
        out_shape=jax.ShapeDtypeStruct((B, N3), jnp.float32),
        grid=(B // NB,),
        in_specs=[
            smem,
            smem,
            pl.BlockSpec((NB, C_IN, IMG * IMG), lambda b: (b, 0, 0)),
            vconst(a1), vconst(a2),
            vconst(S1), vconst(G), vconst(fc1_b), vconst(fc2_wT),
            vconst(fc2_b), vconst(fc3_wT), vconst(fc3_b),
        ],
        out_specs=pl.BlockSpec((NB, N3), lambda b: (b, 0)),
        compiler_params=pltpu.CompilerParams(
            dimension_semantics=("parallel",)),
    )(b1, b2, x_flat, a1, a2, S1, G, fc1_b, fc2_wT, fc2_b, fc3_wT, fc3_b)
    return out


def kernel(w1_taps, b1, w2_taps, b2, S1, G, fc1_b, fc2_wT, fc2_b, fc3_wT,
           fc3_b, x):
    return _lenet_forward(w1_taps, b1, w2_taps, b2, S1, G, fc1_b, fc2_wT,
                          fc2_b, fc3_wT, fc3_b, x)


# speedup vs baseline: 20.3973x; 1.0034x over previous
import numpy as np
import jax
import jax.numpy as jnp
from jax.experimental import pallas as pl
from jax.experimental.pallas import tpu as pltpu

C_IN, C1, C2 = 3, 6, 16
IMG, K = 32, 5
OH1 = IMG - K + 1
W1 = (OH1 - 1) * IMG + OH1
P1 = (W1 - 1) - IMG
POOL1 = 14
OH2 = POOL1 - K + 1
W2 = (OH2 - 1) * POOL1 + OH2
P2 = (W2 - 1) - POOL1
N1, N2, N3 = 120, 84, 10

NB = 512
CH = 120
NCH = 8
KIN = CH + (K - 1) * IMG + K - 1 + 1

_E1 = np.zeros((K * K, KIN, CH), np.float32)
_E2 = np.zeros((K * K, POOL1 * POOL1, W2), np.float32)
for _t in range(K * K):
    _kh, _kw = divmod(_t, K)
    _o1 = _kh * IMG + _kw
    _o2 = _kh * POOL1 + _kw
    for _q in range(CH):
        if _q + _o1 < KIN:
            _E1[_t, _q + _o1, _q] = 1.0
    for _q in range(W2):
        _E2[_t, _q + _o2, _q] = 1.0


def _lenet_block_kernel(b1_ref, b2_ref, x_ref, a1_ref, a2_ref, s1_ref, g_ref,
                        fc1b_ref, fc2w_ref, fc2b_ref, fc3w_ref, fc3b_ref,
                        o_ref):

    ys = []
    for j in range(NCH):
        lo = CH * j
        hi = min(lo + KIN, IMG * IMG)
        kw_ = hi - lo
        acc = None
        for ci in range(C_IN):
            d = jnp.dot(x_ref[:, ci, lo:hi], a1_ref[ci, :kw_, :],
                        preferred_element_type=jnp.float32)
            acc = d if acc is None else acc + d
        ys.append(acc)

    p1 = []
    for co in range(C1):
        y = jnp.concatenate([ys[j][:, co * CH:(co + 1) * CH]
                             for j in range(NCH)], axis=1)
        y = jnp.maximum(y[:, :W1] + b1_ref[co, 0], 0.0)
        m = jnp.maximum(y[:, :-1], y[:, 1:])
        m = jnp.maximum(m[:, :P1], m[:, IMG:IMG + P1])
        p1.append(jnp.dot(m, s1_ref[...],
                          preferred_element_type=jnp.float32))

    y2 = None
    for ci in range(C1):
        d = jnp.dot(p1[ci], a2_ref[ci], preferred_element_type=jnp.float32)
        y2 = d if y2 is None else y2 + d

    h1 = None
    for co in range(C2):
        y = jnp.maximum(y2[:, co * W2:(co + 1) * W2] + b2_ref[co, 0], 0.0)
        m2 = jnp.maximum(y[:, :-1], y[:, 1:])
        m2 = jnp.maximum(m2[:, :P2], m2[:, POOL1:POOL1 + P2])
        d = jnp.dot(m2, g_ref[co], preferred_element_type=jnp.float32)
        h1 = d if h1 is None else h1 + d
    h1 = jnp.maximum(h1 + fc1b_ref[...], 0.0)

    h2 = jnp.maximum(
        jnp.dot(h1, fc2w_ref[...], preferred_element_type=jnp.float32)
        + fc2b_ref[...], 0.0)
    o_ref[...] = jnp.dot(h2, fc3w_ref[...],
                         preferred_element_type=jnp.float32) + fc3b_ref[...]


@jax.jit
def _lenet_forward(w1_taps, b1, w2_taps, b2, S1, G, fc1_b, fc2_wT, fc2_b,
                   fc3_wT, fc3_b, x):
    B = x.shape[0]
    x_flat = x.reshape(B, C_IN, IMG * IMG)

    a1 = jnp.einsum("tkq,toc->ckoq", jnp.asarray(_E1),
                    w1_taps).reshape(C_IN, KIN, C1 * CH)
    a2 = jnp.einsum("tkq,toc->ckoq", jnp.asarray(_E2),
                    w2_taps).reshape(C1, POOL1 * POOL1, C2 * W2)

    def vconst(a):
        return pl.BlockSpec(a.shape, lambda b, _n=a.ndim: (0,) * _n)

    smem = pl.BlockSpec(memory_space=pltpu.SMEM)

    out = pl.pallas_call(
        _lenet_block_kernel,
        out_shape=jax.ShapeDtypeStruct((B, N3), jnp.float32),
        grid=(B // NB,),
        in_specs=[
            smem,
            smem,
            pl.BlockSpec((NB, C_IN, IMG * IMG), lambda b: (b, 0, 0)),
            vconst(a1), vconst(a2),
            vconst(S1), vconst(G), vconst(fc1_b), vconst(fc2_wT),
            vconst(fc2_b), vconst(fc3_wT), vconst(fc3_b),
        ],
        out_specs=pl.BlockSpec((NB, N3), lambda b: (b, 0)),
        compiler_params=pltpu.CompilerParams(
            dimension_semantics=("arbitrary",)),
    )(b1, b2, x_flat, a1, a2, S1, G, fc1_b, fc2_wT, fc2_b, fc3_wT, fc3_b)
    return out


def kernel(w1_taps, b1, w2_taps, b2, S1, G, fc1_b, fc2_wT, fc2_b, fc3_wT,
           fc3_b, x):
    return _lenet_forward(w1_taps, b1, w2_taps, b2, S1, G, fc1_b, fc2_wT,
                          fc2_b, fc3_wT, fc3_b, x)
